# baseline (speedup 1.0000x reference)
import functools

import jax
import jax.numpy as jnp
from jax import lax
from jax.experimental import pallas as pl
from jax.experimental.pallas import tpu as pltpu
from jax.experimental.pallas import tpu_sc as plsc

B = 16384
D = 64
NC = 2
NS = 16
NW = NC * NS
BPW = B // NW
CH = 128
NCH = BPW // CH


def _body(uid_idx, iid_idx, uid_tab, iid_tab, out, idx_u, idx_i, tmp, sem):
    wid = lax.axis_index("s") * NC + lax.axis_index("c")
    base = wid * BPW
    pltpu.sync_copy(uid_idx.at[wid], idx_u)
    pltpu.sync_copy(iid_idx.at[wid], idx_i)
    for t in range(2):
        idx_v = (idx_u, idx_i)[t]
        tab = (uid_tab, iid_tab)[t]
        for j in range(NCH):
            pltpu.async_copy(tab.at[idx_v.at[j]], tmp, sem).wait()
            pltpu.sync_copy(tmp, out.at[pl.ds(base + j * CH, CH), t])


@jax.jit
def _lookup(uid_idx, iid_idx, uid_table, iid_table):
    mesh = plsc.VectorSubcoreMesh(core_axis_name="c", subcore_axis_name="s")
    f = functools.partial(
        pl.kernel,
        mesh=mesh,
        out_type=jax.ShapeDtypeStruct((B, 2, D), jnp.float32),
        scratch_types=[
            pltpu.VMEM((NCH, CH), jnp.int32),
            pltpu.VMEM((NCH, CH), jnp.int32),
            pltpu.VMEM((CH, D), jnp.float32),
            pltpu.SemaphoreType.DMA,
        ],
        compiler_params=pltpu.CompilerParams(use_tc_tiling_on_sc=False),
    )(_body)
    return f(uid_idx, iid_idx, uid_table, iid_table)


def kernel(x, uid_table, iid_table):
    xi = x.astype(jnp.int32)
    uid_idx = xi[:, 0].reshape(NW, NCH, CH)
    iid_idx = xi[:, 1].reshape(NW, NCH, CH)
    return _lookup(uid_idx, iid_idx, uid_table, iid_table)


# speedup vs baseline: 1.0064x; 1.0064x over previous
import functools

import jax
import jax.numpy as jnp
from jax import lax
from jax.experimental import pallas as pl
from jax.experimental.pallas import tpu as pltpu
from jax.experimental.pallas import tpu_sc as plsc

B = 16384
D = 64
NC = 2
NS = 16
NW = NC * NS
BPW = B // NW
CH = 128
NCH = BPW // CH


def _body(uid_idx, iid_idx, uid_tab, iid_tab, out, idx_u, idx_i, rows_u, rows_i, sem):
    wid = lax.axis_index("s") * NC + lax.axis_index("c")
    base = wid * BPW
    pltpu.sync_copy(uid_idx.at[wid], idx_u)
    pltpu.sync_copy(iid_idx.at[wid], idx_i)
    copies = []
    for t in range(2):
        idx_v = (idx_u, idx_i)[t]
        tab = (uid_tab, iid_tab)[t]
        rows = (rows_u, rows_i)[t]
        for j in range(NCH):
            copies.append(pltpu.async_copy(
                tab.at[idx_v.at[j]], rows.at[pl.ds(j * CH, CH)], sem))
    for c in copies:
        c.wait()
    pltpu.sync_copy(rows_u, out.at[pl.ds(base, BPW), 0])
    pltpu.sync_copy(rows_i, out.at[pl.ds(base, BPW), 1])


@jax.jit
def _lookup(uid_idx, iid_idx, uid_table, iid_table):
    mesh = plsc.VectorSubcoreMesh(core_axis_name="c", subcore_axis_name="s")
    f = functools.partial(
        pl.kernel,
        mesh=mesh,
        out_type=jax.ShapeDtypeStruct((B, 2, D), jnp.float32),
        scratch_types=[
            pltpu.VMEM((NCH, CH), jnp.int32),
            pltpu.VMEM((NCH, CH), jnp.int32),
            pltpu.VMEM((BPW, D), jnp.float32),
            pltpu.VMEM((BPW, D), jnp.float32),
            pltpu.SemaphoreType.DMA,
        ],
        compiler_params=pltpu.CompilerParams(use_tc_tiling_on_sc=False),
    )(_body)
    return f(uid_idx, iid_idx, uid_table, iid_table)


def kernel(x, uid_table, iid_table):
    xi = x.astype(jnp.int32)
    uid_idx = xi[:, 0].reshape(NW, NCH, CH)
    iid_idx = xi[:, 1].reshape(NW, NCH, CH)
    return _lookup(uid_idx, iid_idx, uid_table, iid_table)


# speedup vs baseline: 2.0851x; 2.0719x over previous
import functools

import jax
import jax.numpy as jnp
from jax import lax
from jax.experimental import pallas as pl
from jax.experimental.pallas import tpu as pltpu
from jax.experimental.pallas import tpu_sc as plsc

B = 16384
D = 64
NC = 2
NS = 16
NW = NC * NS
BPW = B // NW
CH = 16
NCHUNK = BPW // CH
TROWS = 8


def _scalar(vec, lane_iota, e):
    return jnp.max(jnp.where(lane_iota == e, vec, jnp.int32(-1)))


def _body(xu_h, xv_h, uid_tab, iid_tab, out,
          xu_v, xv_v, tiles_u, tiles_i, rows_c, sem):
    wid = lax.axis_index("s") * NC + lax.axis_index("c")
    base = wid * BPW
    pltpu.sync_copy(xu_h.at[pl.ds(base, BPW)], xu_v)
    pltpu.sync_copy(xv_h.at[pl.ds(base, BPW)], xv_v)
    lane_iota = lax.iota(jnp.int32, 16)

    def chunk(c, _):
        vec_u = xu_v[pl.ds(c * CH, CH)]
        vec_i = xv_v[pl.ds(c * CH, CH)]
        subs = []
        copies = []
        for e in range(CH):
            ru = _scalar(vec_u, lane_iota, e)
            ri = _scalar(vec_i, lane_iota, e)
            subs.append((ru & 7, ri & 7))
            copies.append(pltpu.async_copy(
                uid_tab.at[pl.ds(ru >> 3, 1)], tiles_u.at[pl.ds(e, 1)], sem))
            copies.append(pltpu.async_copy(
                iid_tab.at[pl.ds(ri >> 3, 1)], tiles_i.at[pl.ds(e, 1)], sem))
        for cp in copies:
            cp.wait()
        for e in range(CH):
            su, si = subs[e]
            for k in range(D // 16):
                rows_c[e, 0, pl.ds(16 * k, 16)] = tiles_u[e, su, pl.ds(16 * k, 16)]
                rows_c[e, 1, pl.ds(16 * k, 16)] = tiles_i[e, si, pl.ds(16 * k, 16)]
        pltpu.sync_copy(rows_c, out.at[pl.ds(base + c * CH, CH)])
        return ()

    lax.fori_loop(0, NCHUNK, chunk, (), unroll=False)


@jax.jit
def _lookup(xu, xv, uid_table, iid_table):
    mesh = plsc.VectorSubcoreMesh(core_axis_name="c", subcore_axis_name="s")
    f = functools.partial(
        pl.kernel,
        mesh=mesh,
        out_type=jax.ShapeDtypeStruct((B, 2, D), jnp.float32),
        scratch_types=[
            pltpu.VMEM((BPW,), jnp.int32),
            pltpu.VMEM((BPW,), jnp.int32),
            pltpu.VMEM((CH, TROWS, D), jnp.float32),
            pltpu.VMEM((CH, TROWS, D), jnp.float32),
            pltpu.VMEM((CH, 2, D), jnp.float32),
            pltpu.SemaphoreType.DMA,
        ],
        compiler_params=pltpu.CompilerParams(needs_layout_passes=False),
    )(_body)
    return f(xu, xv, uid_table, iid_table)


def kernel(x, uid_table, iid_table):
    xi = x.astype(jnp.int32)
    ut = uid_table.reshape(125000, TROWS, D)
    it = iid_table[:1000000].reshape(125000, TROWS, D)
    return _lookup(xi[:, 0], xi[:, 1], ut, it)
